# baseline (speedup 1.0000x reference)
import functools

import jax
import jax.numpy as jnp
from jax import lax
from jax.experimental import pallas as pl
from jax.experimental.pallas import tpu as pltpu
from jax.experimental.pallas import tpu_sc as plsc

BATCH = 16384
N_FIELDS = 26
FEATURE_DIM = 1000000
NC = 2
NS = 16
NW = NC * NS
B_PER_W = BATCH // NW
CHUNK = B_PER_W * N_FIELDS
TC_BLK = 2048


@functools.partial(
    pl.kernel,
    out_type=jax.ShapeDtypeStruct((NW, CHUNK), jnp.float32),
    mesh=plsc.VectorSubcoreMesh(core_axis_name="c", subcore_axis_name="s"),
    compiler_params=pltpu.CompilerParams(needs_layout_passes=False),
    scratch_types=[
        pltpu.VMEM((CHUNK,), jnp.int32),
        pltpu.VMEM((CHUNK,), jnp.float32),
        pltpu.SemaphoreType.DMA,
    ],
)
def _sc_gather(idx_hbm, w_hbm, out_hbm, idx_v, rows_v, sem):
    w = lax.axis_index("c") * NS + lax.axis_index("s")
    pltpu.sync_copy(idx_hbm.at[w], idx_v)
    pltpu.async_copy(w_hbm.at[0].at[idx_v], rows_v, sem).wait()
    pltpu.sync_copy(rows_v, out_hbm.at[w])


def _reduce_body(g_ref, v_ref, b_ref, out_ref):
    out_ref[...] = (jnp.sum(g_ref[...] * v_ref[...], axis=1, keepdims=True)
                    + b_ref[0])


_tc_reduce = pl.pallas_call(
    _reduce_body,
    grid=(BATCH // TC_BLK,),
    in_specs=[
        pl.BlockSpec((TC_BLK, N_FIELDS), lambda i: (i, 0)),
        pl.BlockSpec((TC_BLK, N_FIELDS), lambda i: (i, 0)),
        pl.BlockSpec(memory_space=pltpu.SMEM),
    ],
    out_specs=pl.BlockSpec((TC_BLK, 1), lambda i: (i, 0)),
    out_shape=jax.ShapeDtypeStruct((BATCH, 1), jnp.float32),
)


def kernel(feature_idx, feature_vals, W, b):
    idx = feature_idx.astype(jnp.int32).reshape(NW, CHUNK)
    gathered = _sc_gather(idx, W.T).reshape(BATCH, N_FIELDS)
    return _tc_reduce(gathered, feature_vals, b)


# speedup vs baseline: 1.7902x; 1.7902x over previous
import functools

import jax
import jax.numpy as jnp
from jax import lax
from jax.experimental import pallas as pl
from jax.experimental.pallas import tpu as pltpu
from jax.experimental.pallas import tpu_sc as plsc

BATCH = 16384
N_FIELDS = 26
FEATURE_DIM = 1000000
NC = 2
NS = 16
NW = NC * NS
B_PER_W = BATCH // NW
CHUNK = B_PER_W * N_FIELDS
LANES = 16
N_VECS = B_PER_W // LANES
FIELD_GROUPS = ((0, 7), (7, 14), (14, 20), (20, 26))


@functools.partial(
    pl.kernel,
    out_type=jax.ShapeDtypeStruct((BATCH,), jnp.float32),
    mesh=plsc.VectorSubcoreMesh(core_axis_name="c", subcore_axis_name="s"),
    compiler_params=pltpu.CompilerParams(needs_layout_passes=False,
                                         use_tc_tiling_on_sc=True),
    scratch_types=[
        pltpu.VMEM((CHUNK,), jnp.int32),
        pltpu.VMEM((CHUNK,), jnp.float32),
        pltpu.VMEM((CHUNK,), jnp.float32),
        pltpu.VMEM((B_PER_W,), jnp.float32),
        pltpu.SemaphoreType.DMA,
    ],
)
def _sc_linear(idx_hbm, vals_hbm, w_hbm, out_hbm, idx_v, rows_v, vals_v,
               out_v, sem):
    w = lax.axis_index("c") * NS + lax.axis_index("s")
    pltpu.sync_copy(idx_hbm.at[w], idx_v)
    gathers = []
    for lo, hi in FIELD_GROUPS:
        sl = pl.ds(lo * B_PER_W, (hi - lo) * B_PER_W)
        gathers.append(
            pltpu.async_copy(w_hbm.at[0].at[idx_v.at[sl]], rows_v.at[sl],
                             sem))
    pltpu.sync_copy(vals_hbm.at[w], vals_v)

    for k, (lo, hi) in enumerate(FIELD_GROUPS):
        gathers[k].wait()

        def body(s, carry, lo=lo, hi=hi, first=(k == 0)):
            base = s * LANES
            acc = (jnp.zeros((LANES,), jnp.float32) if first
                   else out_v[pl.ds(base, LANES)])
            for j in range(lo, hi):
                off = pl.ds(j * B_PER_W + base, LANES)
                acc = acc + rows_v[off] * vals_v[off]
            out_v[pl.ds(base, LANES)] = acc
            return carry

        lax.fori_loop(0, N_VECS, body, 0)
    pltpu.sync_copy(out_v, out_hbm.at[pl.ds(w * B_PER_W, B_PER_W)])


def kernel(feature_idx, feature_vals, W, b):
    idx = (feature_idx.astype(jnp.int32)
           .reshape(NW, B_PER_W, N_FIELDS).transpose(0, 2, 1)
           .reshape(NW, CHUNK))
    vals = (feature_vals.reshape(NW, B_PER_W, N_FIELDS).transpose(0, 2, 1)
            .reshape(NW, CHUNK))
    out = _sc_linear(idx, vals, W.T)
    return out.reshape(BATCH, 1) + b


# speedup vs baseline: 1.7915x; 1.0007x over previous
import functools

import jax
import jax.numpy as jnp
from jax import lax
from jax.experimental import pallas as pl
from jax.experimental.pallas import tpu as pltpu
from jax.experimental.pallas import tpu_sc as plsc

BATCH = 16384
N_FIELDS = 26
FEATURE_DIM = 1000000
NC = 2
NS = 16
NW = NC * NS
B_PER_W = BATCH // NW
CHUNK = B_PER_W * N_FIELDS
LANES = 16
N_VECS = B_PER_W // LANES
FIELD_GROUPS = ((0, 7), (7, 14), (14, 20), (20, 26))


@functools.partial(
    pl.kernel,
    out_type=jax.ShapeDtypeStruct((BATCH,), jnp.float32),
    mesh=plsc.VectorSubcoreMesh(core_axis_name="c", subcore_axis_name="s"),
    compiler_params=pltpu.CompilerParams(needs_layout_passes=False,
                                         use_tc_tiling_on_sc=True),
    scratch_types=[
        pltpu.VMEM((CHUNK,), jnp.int32),
        pltpu.VMEM((CHUNK,), jnp.float32),
        pltpu.VMEM((CHUNK,), jnp.float32),
        pltpu.VMEM((B_PER_W,), jnp.float32),
        pltpu.SemaphoreType.DMA,
        pltpu.SemaphoreType.DMA,
        pltpu.SemaphoreType.DMA,
        pltpu.SemaphoreType.DMA,
    ],
)
def _sc_linear(idx_hbm, vals_hbm, w_hbm, out_hbm, idx_v, rows_v, vals_v,
               out_v, sem0, sem1, sem2, sem3):
    sems = (sem0, sem1, sem2, sem3)
    w = lax.axis_index("c") * NS + lax.axis_index("s")
    pltpu.sync_copy(idx_hbm.at[w], idx_v)
    gathers = []
    for k, (lo, hi) in enumerate(FIELD_GROUPS):
        sl = pl.ds(lo * B_PER_W, (hi - lo) * B_PER_W)
        gathers.append(
            pltpu.async_copy(w_hbm.at[0].at[idx_v.at[sl]], rows_v.at[sl],
                             sems[k]))
    pltpu.sync_copy(vals_hbm.at[w], vals_v)

    for k, (lo, hi) in enumerate(FIELD_GROUPS):
        gathers[k].wait()

        def body(s, carry, lo=lo, hi=hi, first=(k == 0)):
            base = s * LANES
            acc = (jnp.zeros((LANES,), jnp.float32) if first
                   else out_v[pl.ds(base, LANES)])
            for j in range(lo, hi):
                off = pl.ds(j * B_PER_W + base, LANES)
                acc = acc + rows_v[off] * vals_v[off]
            out_v[pl.ds(base, LANES)] = acc
            return carry

        lax.fori_loop(0, N_VECS, body, 0)
    pltpu.sync_copy(out_v, out_hbm.at[pl.ds(w * B_PER_W, B_PER_W)])


def kernel(feature_idx, feature_vals, W, b):
    idx = (feature_idx.astype(jnp.int32)
           .reshape(NW, B_PER_W, N_FIELDS).transpose(0, 2, 1)
           .reshape(NW, CHUNK))
    vals = (feature_vals.reshape(NW, B_PER_W, N_FIELDS).transpose(0, 2, 1)
            .reshape(NW, CHUNK))
    out = _sc_linear(idx, vals, W.T)
    return out.reshape(BATCH, 1) + b


# speedup vs baseline: 1.8290x; 1.0209x over previous
import functools

import jax
import jax.numpy as jnp
from jax import lax
from jax.experimental import pallas as pl
from jax.experimental.pallas import tpu as pltpu
from jax.experimental.pallas import tpu_sc as plsc

BATCH = 16384
N_FIELDS = 26
FEATURE_DIM = 1000000
NC = 2
NS = 16
NW = NC * NS
B_PER_W = BATCH // NW
CHUNK = B_PER_W * N_FIELDS
LANES = 16
N_VECS = B_PER_W // LANES
FIELD_GROUPS = ((0, 7), (7, 14), (14, 20), (20, 26))


@functools.partial(
    pl.kernel,
    out_type=jax.ShapeDtypeStruct((BATCH,), jnp.float32),
    mesh=plsc.VectorSubcoreMesh(core_axis_name="c", subcore_axis_name="s"),
    compiler_params=pltpu.CompilerParams(needs_layout_passes=False,
                                         use_tc_tiling_on_sc=True),
    scratch_types=[
        pltpu.VMEM((CHUNK,), jnp.int32),
        pltpu.VMEM((CHUNK,), jnp.float32),
        pltpu.VMEM((CHUNK,), jnp.float32),
        pltpu.VMEM((B_PER_W,), jnp.float32),
        pltpu.SemaphoreType.DMA,
        pltpu.SemaphoreType.DMA,
        pltpu.SemaphoreType.DMA,
        pltpu.SemaphoreType.DMA,
    ],
)
def _sc_linear(idx_hbm, vals_hbm, w_hbm, out_hbm, idx_v, rows_v, vals_v,
               out_v, sem0, sem1, sem2, sem3):
    sems = (sem0, sem1, sem2, sem3)
    w = lax.axis_index("c") * NS + lax.axis_index("s")
    gathers = []
    for k, (lo, hi) in enumerate(FIELD_GROUPS):
        sl = pl.ds(lo * B_PER_W, (hi - lo) * B_PER_W)
        pltpu.sync_copy(idx_hbm.at[w, sl], idx_v.at[sl])
        gathers.append(
            pltpu.async_copy(w_hbm.at[0].at[idx_v.at[sl]], rows_v.at[sl],
                             sems[k]))
    pltpu.sync_copy(vals_hbm.at[w], vals_v)

    for k, (lo, hi) in enumerate(FIELD_GROUPS):
        gathers[k].wait()

        def body(s, carry, lo=lo, hi=hi, first=(k == 0)):
            base = s * LANES
            acc = (jnp.zeros((LANES,), jnp.float32) if first
                   else out_v[pl.ds(base, LANES)])
            for j in range(lo, hi):
                off = pl.ds(j * B_PER_W + base, LANES)
                acc = acc + rows_v[off] * vals_v[off]
            out_v[pl.ds(base, LANES)] = acc
            return carry

        lax.fori_loop(0, N_VECS, body, 0)
    pltpu.sync_copy(out_v, out_hbm.at[pl.ds(w * B_PER_W, B_PER_W)])


def kernel(feature_idx, feature_vals, W, b):
    idx = (feature_idx.astype(jnp.int32)
           .reshape(NW, B_PER_W, N_FIELDS).transpose(0, 2, 1)
           .reshape(NW, CHUNK))
    vals = (feature_vals.reshape(NW, B_PER_W, N_FIELDS).transpose(0, 2, 1)
            .reshape(NW, CHUNK))
    out = _sc_linear(idx, vals, W.T)
    return out.reshape(BATCH, 1) + b
